# baseline (speedup 1.0000x reference)
import functools

import jax
import jax.numpy as jnp
from jax import lax
from jax.experimental import pallas as pl
from jax.experimental.pallas import tpu as pltpu
from jax.experimental.pallas import tpu_sc as plsc

F = 64
S = 200
B = 4096
NC = 2
NS = 16
NW = NC * NS
BB = B // NW
LANES = 16
NJ = BB // LANES


def _emb_body(idsT_hbm, table_hbm, pos_hbm, out_hbm,
              idx_all, pos_v, r0, r1, o0, o1, g0, g1, w0, w1):
    rows = (r0, r1)
    obuf = (o0, o1)
    gsem = (g0, g1)
    osem = (w0, w1)
    wid = lax.axis_index("s") * NC + lax.axis_index("c")
    col0 = wid * BB
    pltpu.sync_copy(pos_hbm, pos_v)
    pltpu.sync_copy(idsT_hbm.at[:, pl.ds(col0, BB)], idx_all)

    lane = lax.iota(jnp.int32, LANES)
    rvecs = [lane + (j * LANES) for j in range(NJ)]
    fcols = [jnp.full((LANES,), f, jnp.int32) for f in range(F)]

    def gather_start(s, b):
        pltpu.async_copy(table_hbm.at[idx_all.at[s]], rows[b], gsem[b])

    def gather_wait(s, b):
        pltpu.make_async_copy(table_hbm.at[idx_all.at[s]], rows[b],
                              gsem[b]).wait()

    def write_start(s, b):
        pltpu.async_copy(obuf[b], out_hbm.at[s, :, pl.ds(col0, BB)], osem[b])

    def write_wait(s, b):
        pltpu.make_async_copy(obuf[b], out_hbm.at[s, :, pl.ds(col0, BB)],
                              osem[b]).wait()

    gather_start(0, 0)

    def outer(i, carry):
        for b in range(2):
            s = i * 2 + b
            gather_wait(s, b)

            @pl.when(s + 1 < S)
            def _pref():
                gather_start(s + 1, 1 - b)

            @pl.when(s >= 2)
            def _drain():
                write_wait(s - 2, b)

            pv = [pos_v[s, pl.ds(fb * LANES, LANES)]
                  for fb in range(F // LANES)]
            for f in range(F):
                p = pv[f // LANES][f % LANES]
                for j in range(NJ):
                    v = plsc.load_gather(rows[b], [rvecs[j], fcols[f]])
                    obuf[b][f, pl.ds(j * LANES, LANES)] = v + p

            write_start(s, b)
        return carry

    lax.fori_loop(0, S // 2, outer, 0)
    write_wait(S - 2, 0)
    write_wait(S - 1, 1)


_emb = functools.partial(
    pl.kernel,
    out_type=jax.ShapeDtypeStruct((S, F, B), jnp.float32),
    mesh=plsc.VectorSubcoreMesh(core_axis_name="c", subcore_axis_name="s"),
    scratch_types=[
        pltpu.VMEM((S, BB), jnp.int32),
        pltpu.VMEM((S, 2 * F), jnp.float32),
        pltpu.VMEM((BB, 2 * F), jnp.float32),
        pltpu.VMEM((BB, 2 * F), jnp.float32),
        pltpu.VMEM((F, BB), jnp.float32),
        pltpu.VMEM((F, BB), jnp.float32),
    ] + [pltpu.SemaphoreType.DMA for _ in range(4)],
    compiler_params=pltpu.CompilerParams(use_tc_tiling_on_sc=True,
                                         needs_layout_passes=False),
)(_emb_body)


def kernel(input_ids, input_embedding_weight, position_embedding):
    idsT = input_ids.astype(jnp.int32).T
    table128 = jnp.pad(input_embedding_weight, ((0, 0), (0, F)))
    pos128 = jnp.pad(position_embedding, ((0, 0), (0, F)))
    out_sfb = _emb(idsT, table128, pos128)
    return out_sfb.transpose(2, 0, 1)


# speedup vs baseline: 1.5636x; 1.5636x over previous
import functools

import jax
import jax.numpy as jnp
from jax import lax
from jax.experimental import pallas as pl
from jax.experimental.pallas import tpu as pltpu
from jax.experimental.pallas import tpu_sc as plsc

F = 64
S = 200
B = 4096
NC = 2
NS = 16
NW = NC * NS
BB = B // NW
LANES = 16
NJ = BB // LANES


def _emb_body(idsT_hbm, table_hbm, pos_hbm, out_hbm,
              idx_all, pos_v, r0, r1, o0, o1, g0, g1, w0, w1):
    rows = (r0, r1)
    obuf = (o0, o1)
    gsem = (g0, g1)
    osem = (w0, w1)
    wid = lax.axis_index("s") * NC + lax.axis_index("c")
    col0 = wid * BB
    pltpu.sync_copy(pos_hbm, pos_v)
    pltpu.sync_copy(idsT_hbm.at[:, pl.ds(col0, BB)], idx_all)

    lane = lax.iota(jnp.int32, LANES)
    rvecs = [lane + (j * LANES) for j in range(NJ)]
    fcols = [jnp.full((LANES,), f, jnp.int32) for f in range(F)]

    def gather_start(s, b):
        pltpu.async_copy(table_hbm.at[idx_all.at[s]], rows[b], gsem[b])

    def gather_wait(s, b):
        pltpu.make_async_copy(table_hbm.at[idx_all.at[s]], rows[b],
                              gsem[b]).wait()

    def write_start(s, b):
        pltpu.async_copy(obuf[b], out_hbm.at[s, :, pl.ds(col0, BB)], osem[b])

    def write_wait(s, b):
        pltpu.make_async_copy(obuf[b], out_hbm.at[s, :, pl.ds(col0, BB)],
                              osem[b]).wait()

    gather_start(0, 0)

    def outer(i, carry):
        for b in range(2):
            s = i * 2 + b
            gather_wait(s, b)

            @pl.when(s + 1 < S)
            def _pref():
                gather_start(s + 1, 1 - b)

            @pl.when(s >= 2)
            def _drain():
                write_wait(s - 2, b)

            pv = [pos_v[s, pl.ds(fb * LANES, LANES)]
                  for fb in range(F // LANES)]

            @plsc.parallel_loop(0, BB, unroll=4)
            def _addpos(r):
                for fb in range(F // LANES):
                    plsc.addupdate(rows[b].at[r, pl.ds(fb * LANES, LANES)],
                                   pv[fb])

            @plsc.parallel_loop(0, F, unroll=4)
            def _tr(f):
                fcol = jnp.zeros((LANES,), jnp.int32) + f
                for j in range(NJ):
                    v = plsc.load_gather(rows[b], [rvecs[j], fcol])
                    obuf[b][f, pl.ds(j * LANES, LANES)] = v

            write_start(s, b)
        return carry

    lax.fori_loop(0, S // 2, outer, 0)
    write_wait(S - 2, 0)
    write_wait(S - 1, 1)


_emb = functools.partial(
    pl.kernel,
    out_type=jax.ShapeDtypeStruct((S, F, B), jnp.float32),
    mesh=plsc.VectorSubcoreMesh(core_axis_name="c", subcore_axis_name="s"),
    scratch_types=[
        pltpu.VMEM((S, BB), jnp.int32),
        pltpu.VMEM((S, 2 * F), jnp.float32),
        pltpu.VMEM((BB, 2 * F), jnp.float32),
        pltpu.VMEM((BB, 2 * F), jnp.float32),
        pltpu.VMEM((F, BB), jnp.float32),
        pltpu.VMEM((F, BB), jnp.float32),
    ] + [pltpu.SemaphoreType.DMA for _ in range(4)],
    compiler_params=pltpu.CompilerParams(use_tc_tiling_on_sc=True,
                                         needs_layout_passes=False),
)(_emb_body)


def kernel(input_ids, input_embedding_weight, position_embedding):
    idsT = input_ids.astype(jnp.int32).T
    table128 = jnp.pad(input_embedding_weight, ((0, 0), (0, F)))
    pos128 = jnp.pad(position_embedding, ((0, 0), (0, F)))
    out_sfb = _emb(idsT, table128, pos128)
    return out_sfb.transpose(2, 0, 1)


# speedup vs baseline: 1.6282x; 1.0413x over previous
import functools

import jax
import jax.numpy as jnp
from jax import lax
from jax.experimental import pallas as pl
from jax.experimental.pallas import tpu as pltpu
from jax.experimental.pallas import tpu_sc as plsc

F = 64
S = 200
SP = 256
B = 4096
NC = 2
NS = 16
NW = NC * NS
BB = B // NW
LANES = 16
NJ = BB // LANES


def _emb_body(idsT_hbm, table_hbm, posT_hbm, out_hbm,
              idx_all, posT_v, r0, r1, o0, o1, g0, g1, w0, w1):
    rows = (r0, r1)
    obuf = (o0, o1)
    gsem = (g0, g1)
    osem = (w0, w1)
    wid = lax.axis_index("s") * NC + lax.axis_index("c")
    col0 = wid * BB
    pltpu.sync_copy(posT_hbm, posT_v)
    pltpu.sync_copy(idsT_hbm.at[:, pl.ds(col0, BB)], idx_all)

    lane = lax.iota(jnp.int32, LANES)
    rvecs = [lane + (j * LANES) for j in range(NJ)]
    zero = jnp.zeros((LANES,), jnp.int32)

    def gather_start(s, b):
        pltpu.async_copy(table_hbm.at[idx_all.at[s]], rows[b], gsem[b])

    def gather_wait(s, b):
        pltpu.make_async_copy(table_hbm.at[idx_all.at[s]], rows[b],
                              gsem[b]).wait()

    def write_start(s, b):
        pltpu.async_copy(obuf[b], out_hbm.at[s, :, pl.ds(col0, BB)], osem[b])

    def write_wait(s, b):
        pltpu.make_async_copy(obuf[b], out_hbm.at[s, :, pl.ds(col0, BB)],
                              osem[b]).wait()

    gather_start(0, 0)

    def outer(i, carry):
        for b in range(2):
            s = i * 2 + b
            gather_wait(s, b)

            @pl.when(s + 1 < S)
            def _pref():
                gather_start(s + 1, 1 - b)

            @pl.when(s >= 2)
            def _drain():
                write_wait(s - 2, b)

            scol = zero + s

            @plsc.parallel_loop(0, F, unroll=8)
            def _tr(f):
                fcol = zero + f
                p = plsc.load_gather(posT_v, [fcol, scol])
                for j in range(NJ):
                    v = plsc.load_gather(rows[b], [rvecs[j], fcol])
                    obuf[b][f, pl.ds(j * LANES, LANES)] = v + p

            write_start(s, b)
        return carry

    lax.fori_loop(0, S // 2, outer, 0)
    write_wait(S - 2, 0)
    write_wait(S - 1, 1)


_emb = functools.partial(
    pl.kernel,
    out_type=jax.ShapeDtypeStruct((S, F, B), jnp.float32),
    mesh=plsc.VectorSubcoreMesh(core_axis_name="c", subcore_axis_name="s"),
    scratch_types=[
        pltpu.VMEM((S, BB), jnp.int32),
        pltpu.VMEM((F, SP), jnp.float32),
        pltpu.VMEM((BB, 2 * F), jnp.float32),
        pltpu.VMEM((BB, 2 * F), jnp.float32),
        pltpu.VMEM((F, BB), jnp.float32),
        pltpu.VMEM((F, BB), jnp.float32),
    ] + [pltpu.SemaphoreType.DMA for _ in range(4)],
    compiler_params=pltpu.CompilerParams(use_tc_tiling_on_sc=True,
                                         needs_layout_passes=False),
)(_emb_body)


def kernel(input_ids, input_embedding_weight, position_embedding):
    idsT = input_ids.astype(jnp.int32).T
    table128 = jnp.pad(input_embedding_weight, ((0, 0), (0, F)))
    posT = jnp.pad(position_embedding.T, ((0, 0), (0, SP - S)))
    out_sfb = _emb(idsT, table128, posT)
    return out_sfb.transpose(2, 0, 1)
